# Initial kernel scaffold; baseline (speedup 1.0000x reference)
import functools

import jax
import jax.numpy as jnp
from jax import lax
from jax.experimental import pallas as pl
from jax.experimental.pallas import tpu as pltpu
from jax.experimental.pallas import tpu_sc as plsc

NC = 2
NS = 16
NW = NC * NS
LANES = 16
EPS = 1e-5
NUM_GRAPHS = 64


def _encoder_body(x_ref, w_ref, b_ref, out_ref):
    out_ref[...] = (
        jnp.dot(x_ref[...], w_ref[...], preferred_element_type=jnp.float32)
        + b_ref[...]
    )


def _encode(x, w, b):
    n = x.shape[0]
    h = w.shape[1]
    return pl.pallas_call(
        _encoder_body,
        out_shape=jax.ShapeDtypeStruct((n, h), jnp.float32),
    )(x, w, b.reshape(1, h))


def _dense_body(h_ref, pp_ref, eap_ref, batch_ref, wn_ref, bn_ref, wc_ref,
                w1a_ref, w1b_ref, b1_ref, w2_ref, b2_ref, g_ref, bt_ref,
                out_ref):
    f32 = jnp.float32
    n = h_ref.shape[0]
    g = NUM_GRAPHS
    h = h_ref[...]
    p = pp_ref[0] + pp_ref[1]
    eax = eap_ref[0] + eap_ref[1]

    hn = jnp.dot(h, wn_ref[...], preferred_element_type=f32) + bn_ref[...]
    aggr = (
        jnp.dot(p, wn_ref[...], preferred_element_type=f32)
        + jnp.dot(eax, wc_ref[...], preferred_element_type=f32)
    )
    t = jnp.maximum(
        jnp.dot(hn, w1a_ref[...], preferred_element_type=f32)
        + jnp.dot(aggr, w1b_ref[...], preferred_element_type=f32)
        + b1_ref[...],
        0.0,
    )
    out = jnp.dot(t, w2_ref[...], preferred_element_type=f32) + b2_ref[...]

    oh = (batch_ref[...] == lax.broadcasted_iota(jnp.int32, (n, g), 1))
    oh = oh.astype(f32)
    dnums_t = (((0,), (0,)), ((), ()))
    cnt = lax.dot_general(oh, jnp.ones((n, 1), f32), dnums_t,
                          preferred_element_type=f32)
    sums = lax.dot_general(oh, out, dnums_t,
                           preferred_element_type=f32)
    sums2 = lax.dot_general(oh, out * out, dnums_t,
                            preferred_element_type=f32)
    rcnt = 1.0 / jnp.maximum(cnt, 1.0)
    mean = sums * rcnt
    var = sums2 * rcnt - mean * mean
    mb = jnp.dot(oh, mean, preferred_element_type=f32)
    vb = jnp.dot(oh, var, preferred_element_type=f32)
    xnorm = (out - mb) * lax.rsqrt(vb + EPS)
    outn = g_ref[...] * xnorm + bt_ref[...]
    out_ref[...] = jnp.maximum(outn, 0.0) + h


def _dense_layer(h, pp, eap, batch2, wn, bn, wc, w1a, w1b, b1, w2, b2, gm, bt):
    n, hh = h.shape
    return pl.pallas_call(
        _dense_body,
        out_shape=jax.ShapeDtypeStruct((n, hh), jnp.float32),
    )(h, pp, eap, batch2, wn, bn, wc, w1a, w1b, b1, w2, b2, gm, bt)


def _pick_chunk(epw):
    for ch in range(128, 0, -8):
        if epw % ch == 0:
            return ch
    return 8


def _make_sc_edge(n, h, e, with_ea, ed2):
    epw = e // NW
    ch = _pick_chunk(epw)
    nck = epw // ch
    rpt = n // NS
    rc = rpt
    while rc * h * 4 > 72 * 1024:
        rc //= 5
    ncp = rpt // rc

    mesh = plsc.VectorSubcoreMesh(core_axis_name="c", subcore_axis_name="s",
                                  num_cores=NC, num_subcores=NS)
    out_type = [jax.ShapeDtypeStruct((NC, n, h), jnp.float32)]
    scratch = [
        pltpu.VMEM((nck, ch), jnp.int32),
        pltpu.VMEM((nck, ch), jnp.int32),
        pltpu.VMEM((ch, h), jnp.float32),
        pltpu.VMEM((rc, h), jnp.float32),
        pltpu.VMEM_SHARED((n, h), jnp.float32),
        pltpu.SemaphoreType.DMA,
    ]
    if with_ea:
        out_type.append(jax.ShapeDtypeStruct((NC, n, ed2), jnp.float32))
        scratch += [
            pltpu.VMEM((ch, ed2), jnp.float32),
            pltpu.VMEM((rc, ed2), jnp.float32),
            pltpu.VMEM_SHARED((n, ed2), jnp.float32),
        ]

    def body(*args):
        if with_ea:
            (h_hbm, src_hbm, dst_hbm, ea_hbm, out_p, out_ea,
             src_v, dst_v, rows_v, zbuf, acc_sh, gsem,
             ea_v, zbuf_e, ea_sh) = args
        else:
            (h_hbm, src_hbm, dst_hbm, out_p,
             src_v, dst_v, rows_v, zbuf, acc_sh, gsem) = args
        c = lax.axis_index("c")
        s = lax.axis_index("s")
        wid = c * NS + s
        base = s * rpt

        def zrow(i, _):
            for j in range(h // LANES):
                zbuf[i, pl.ds(j * LANES, LANES)] = jnp.zeros((LANES,),
                                                             jnp.float32)
            if with_ea:
                for j in range(ed2 // LANES):
                    zbuf_e[i, pl.ds(j * LANES, LANES)] = jnp.zeros(
                        (LANES,), jnp.float32)
            return 0

        lax.fori_loop(0, rc, zrow, 0)
        for k in range(ncp):
            pltpu.sync_copy(zbuf, acc_sh.at[pl.ds(base + k * rc, rc)])
            if with_ea:
                pltpu.sync_copy(zbuf_e, ea_sh.at[pl.ds(base + k * rc, rc)])
        plsc.subcore_barrier()

        pltpu.sync_copy(src_hbm.at[wid], src_v)
        pltpu.sync_copy(dst_hbm.at[wid], dst_v)

        def chunk(j, _):
            cp = pltpu.async_copy(h_hbm.at[src_v.at[j]], rows_v, gsem)
            if with_ea:
                pltpu.sync_copy(ea_hbm.at[wid, j], ea_v)
                pltpu.sync_copy(ea_v, ea_sh.at[dst_v.at[j]], add=True)
            cp.wait()
            pltpu.sync_copy(rows_v, acc_sh.at[dst_v.at[j]], add=True)
            return 0

        lax.fori_loop(0, nck, chunk, 0)
        plsc.subcore_barrier()

        for k in range(ncp):
            r = base + k * rc
            pltpu.sync_copy(acc_sh.at[pl.ds(r, rc)], zbuf)
            pltpu.sync_copy(zbuf, out_p.at[c, pl.ds(r, rc)])
            if with_ea:
                pltpu.sync_copy(ea_sh.at[pl.ds(r, rc)], zbuf_e)
                pltpu.sync_copy(zbuf_e, out_ea.at[c, pl.ds(r, rc)])

    return pl.kernel(body, out_type=out_type, mesh=mesh,
                     scratch_types=scratch), nck, ch


def kernel(x, edge_index, edge_attr, batch, W_enc, b_enc, Wn, bn, We, be,
           W1, b1, W2, b2, gamma, beta):
    n, _ = x.shape
    e = edge_index.shape[1]
    hh = W_enc.shape[1]
    nl = Wn.shape[0]
    ed = edge_attr.shape[1]
    ed2 = 32

    sc0, nck, ch = _make_sc_edge(n, hh, e, True, ed2)
    scp, _, _ = _make_sc_edge(n, hh, e, False, ed2)

    src = edge_index[0].astype(jnp.int32).reshape(NW, nck, ch)
    dst = edge_index[1].astype(jnp.int32).reshape(NW, nck, ch)
    ea_aug = jnp.concatenate(
        [edge_attr.astype(jnp.float32),
         jnp.ones((e, 1), jnp.float32),
         jnp.zeros((e, ed2 - ed - 1), jnp.float32)], axis=1,
    ).reshape(NW, nck, ch, ed2)
    batch2 = batch.astype(jnp.int32).reshape(n, 1)

    wc = jnp.concatenate(
        [We, (bn + be)[:, None, :],
         jnp.zeros((nl, ed2 - ed - 1, hh), jnp.float32)], axis=1)

    h = _encode(x, W_enc, b_enc)
    eap = None
    for l in range(nl):
        if l == 0:
            pp, eap = sc0(h, src, dst, ea_aug)
        else:
            pp = scp(h, src, dst)
        h = _dense_layer(
            h, pp, eap, batch2,
            Wn[l], bn[l].reshape(1, hh), wc[l],
            W1[l, :hh], W1[l, hh:], b1[l].reshape(1, hh),
            W2[l], b2[l].reshape(1, hh),
            gamma[l].reshape(1, hh), beta[l].reshape(1, hh))
    return h


# speedup vs baseline: 2.9900x; 2.9900x over previous
import jax
import jax.numpy as jnp
from jax import lax
from jax.experimental import pallas as pl
from jax.experimental.pallas import tpu as pltpu
from jax.experimental.pallas import tpu_sc as plsc

NC = 2
NS = 16
NW = NC * NS
LANES = 16
EPS = 1e-5
NUM_GRAPHS = 64
RB = 80


def _encoder_body(x_ref, w_ref, b_ref, out_ref):
    out_ref[...] = (
        jnp.dot(x_ref[...], w_ref[...], preferred_element_type=jnp.float32,
                precision=lax.Precision.HIGHEST)
        + b_ref[...]
    )


def _encode(x, w, b):
    n = x.shape[0]
    h = w.shape[1]
    return pl.pallas_call(
        _encoder_body,
        out_shape=jax.ShapeDtypeStruct((n, h), jnp.float32),
    )(x, w, b.reshape(1, h))


def _make_dense_body():
    def _dense_body(h_ref, pp_ref, ea_ref, batch_ref, wn_ref,
                    bn_ref, wc_ref, w1a_ref, w1b_ref, b1_ref, w2_ref,
                    b2_ref, g_ref, bt_ref, out_ref):
        f32 = jnp.float32
        n = h_ref.shape[0]
        g = NUM_GRAPHS
        h = h_ref[...]
        p = jnp.concatenate([pp_ref[0, :, :], pp_ref[1, :, :]], axis=0)
        eadg = jnp.concatenate([ea_ref[0, :, :], ea_ref[1, :, :]], axis=0)

        hn = jnp.dot(h, wn_ref[...], preferred_element_type=f32,
                     precision=lax.Precision.HIGHEST) + bn_ref[...]
        aggr = (
            jnp.dot(p, wn_ref[...], preferred_element_type=f32,
                    precision=lax.Precision.HIGHEST)
            + jnp.dot(eadg[:, :wc_ref.shape[0]], wc_ref[...],
                      preferred_element_type=f32,
                      precision=lax.Precision.HIGHEST)
        )
        t = jnp.maximum(
            jnp.dot(hn, w1a_ref[...], preferred_element_type=f32,
                    precision=lax.Precision.HIGHEST)
            + jnp.dot(aggr, w1b_ref[...], preferred_element_type=f32,
                      precision=lax.Precision.HIGHEST)
            + b1_ref[...],
            0.0,
        )
        out = jnp.dot(t, w2_ref[...], preferred_element_type=f32,
                      precision=lax.Precision.HIGHEST) + b2_ref[...]

        oh = (batch_ref[...] == lax.broadcasted_iota(jnp.int32, (n, g), 1))
        oh = oh.astype(f32)
        dnums_t = (((0,), (0,)), ((), ()))
        cnt = lax.dot_general(oh, jnp.ones((n, 1), f32), dnums_t,
                              preferred_element_type=f32)
        sums = lax.dot_general(oh, out, dnums_t,
                               preferred_element_type=f32)
        rcnt = 1.0 / jnp.maximum(cnt, 1.0)
        mean = sums * rcnt
        mb = jnp.dot(oh, mean, preferred_element_type=f32)
        ctr = out - mb
        sums2 = lax.dot_general(oh, ctr * ctr, dnums_t,
                                preferred_element_type=f32)
        vb = jnp.dot(oh, sums2 * rcnt, preferred_element_type=f32)
        xnorm = ctr * lax.rsqrt(vb + EPS)
        out_ref[...] = jnp.maximum(g_ref[...] * xnorm + bt_ref[...], 0.0) + h
    return _dense_body


def _dense_layer(h, pp, eadg, batch2, wn, bn, wc, w1a, w1b, b1, w2,
                 b2, gm, bt):
    n, hh = h.shape
    return pl.pallas_call(
        _make_dense_body(),
        out_shape=jax.ShapeDtypeStruct((n, hh), jnp.float32),
    )(h, pp, eadg, batch2, wn, bn, wc, w1a, w1b, b1, w2, b2, gm, bt)


def _pick_chunk(epw):
    for ch in range(128, 0, -8):
        if epw % ch == 0:
            return ch
    return 8


def _make_sc_edge(np_, h, e):
    half = np_ // NC
    ept = e // NS
    ch = _pick_chunk(ept)
    nck = ept // ch
    rpt = half // NS
    ncp = rpt // RB

    mesh = plsc.VectorSubcoreMesh(core_axis_name="c", subcore_axis_name="s",
                                  num_cores=NC, num_subcores=NS)
    out_type = [jax.ShapeDtypeStruct((NC, half, h), jnp.float32)]
    scratch = [
        pltpu.VMEM((nck, ch), jnp.int32),
        pltpu.VMEM((nck, ch), jnp.int32),
        pltpu.VMEM((ch, h), jnp.float32),
        pltpu.VMEM((RB, h), jnp.float32),
        pltpu.VMEM_SHARED((half + NS, h), jnp.float32),
        pltpu.SemaphoreType.DMA,
    ]

    def body(h_hbm, src_hbm, dstr_hbm, out_p,
             src_v, dst_v, rows_v, zbuf, acc_sh, gsem):
        c = lax.axis_index("c")
        s = lax.axis_index("s")
        base = s * rpt

        def zrow(i, _):
            for j in range(h // LANES):
                zbuf[i, pl.ds(j * LANES, LANES)] = jnp.zeros((LANES,),
                                                             jnp.float32)
            return 0

        lax.fori_loop(0, RB, zrow, 0)
        for k in range(ncp):
            pltpu.sync_copy(zbuf, acc_sh.at[pl.ds(base + k * RB, RB)])
        plsc.subcore_barrier()

        pltpu.sync_copy(src_hbm.at[s], src_v)
        pltpu.sync_copy(dstr_hbm.at[c * NS + s], dst_v)

        def chunk(j, _):
            cp = pltpu.async_copy(h_hbm.at[src_v.at[j]], rows_v, gsem)
            cp.wait()
            pltpu.sync_copy(rows_v, acc_sh.at[dst_v.at[j]], add=True)
            return 0

        lax.fori_loop(0, nck, chunk, 0)
        plsc.subcore_barrier()

        for k in range(ncp):
            r = base + k * RB
            pltpu.sync_copy(acc_sh.at[pl.ds(r, RB)], zbuf)
            pltpu.sync_copy(zbuf, out_p.at[c, pl.ds(r, RB)])

    return pl.kernel(body, out_type=out_type, mesh=mesh,
                     scratch_types=scratch), nck, ch


def kernel(x, edge_index, edge_attr, batch, W_enc, b_enc, Wn, bn, We, be,
           W1, b1, W2, b2, gamma, beta):
    n, _ = x.shape
    e = edge_index.shape[1]
    hh = W_enc.shape[1]
    nl = Wn.shape[0]
    ed = edge_attr.shape[1]

    unit = NS * RB
    np_ = ((n + unit - 1) // unit) * unit

    scp, nckp, chp = _make_sc_edge(np_, hh, e)

    src = edge_index[0].astype(jnp.int32)
    dst = edge_index[1].astype(jnp.int32)
    half = np_ // NC
    src3 = src.reshape(NS, nckp, chp)
    dsts = dst.reshape(NS, nckp, chp)
    trash = half + jnp.arange(NS, dtype=jnp.int32).reshape(NS, 1, 1)
    d0 = jnp.where(dsts < half, dsts, trash)
    d1 = jnp.where(dsts >= half, dsts - half, trash)
    dstr = jnp.stack([d0, d1]).reshape(NC * NS, nckp, chp)

    ea_aug = jnp.concatenate(
        [edge_attr.astype(jnp.float32),
         jnp.ones((e, 1), jnp.float32),
         jnp.zeros((e, hh - ed - 1), jnp.float32)], axis=1)
    eidx = jnp.arange(e, dtype=jnp.int32).reshape(NS, nckp, chp)

    xp = jnp.pad(x.astype(jnp.float32), ((0, np_ - n), (0, 0)))
    batch2 = jnp.pad(batch.astype(jnp.int32), (0, np_ - n),
                     constant_values=NUM_GRAPHS).reshape(np_, 1)

    h = _encode(xp, W_enc, b_enc)
    (eadg,) = scp(ea_aug, eidx, dstr)
    wc = jnp.concatenate(
        [We, (bn + be)[:, None, :],
         jnp.zeros((nl, 32 - ed - 1, hh), jnp.float32)], axis=1)
    for l in range(nl):
        (pp,) = scp(h, src3, dstr)
        h = _dense_layer(
            h, pp, eadg, batch2,
            Wn[l], bn[l].reshape(1, hh), wc[l],
            W1[l, :hh], W1[l, hh:], b1[l].reshape(1, hh),
            W2[l], b2[l].reshape(1, hh),
            gamma[l].reshape(1, hh), beta[l].reshape(1, hh))
    return h[:n]
